# baseline (speedup 1.0000x reference)
import functools

import jax
import jax.numpy as jnp
from jax.experimental import pallas as pl
from jax.experimental.pallas import tpu as pltpu


def _se_ring_kernel(x_hbm, w1_ref, w2_ref, o_hbm,
                    in_buf, out_buf, in_sem, out_sem, *, n, k, tb, inv_s):
    def in_copy(i, slot):
        return pltpu.make_async_copy(
            x_hbm.at[pl.ds(i * tb, tb)], in_buf.at[slot], in_sem.at[slot])

    def out_copy(i, slot):
        return pltpu.make_async_copy(
            out_buf.at[slot], o_hbm.at[pl.ds(i * tb, tb)], out_sem.at[slot])

    for j in range(min(k, n)):
        in_copy(j, j).start()

    def body(i, carry):
        slot = jax.lax.rem(i, k)
        in_copy(i, slot).wait()
        xv = in_buf[slot]
        se = jnp.sum(xv.astype(jnp.float32), axis=-1) * inv_s
        h = jnp.maximum(
            jnp.dot(se, w1_ref[...].astype(jnp.float32),
                    preferred_element_type=jnp.float32), 0.0)
        g = jax.nn.sigmoid(
            jnp.dot(h, w2_ref[...].astype(jnp.float32),
                    preferred_element_type=jnp.float32))

        @pl.when(i >= k)
        def _():
            out_copy(i - k, slot).wait()

        out_buf[slot] = xv * g[:, :, None].astype(xv.dtype)
        out_copy(i, slot).start()

        @pl.when(i + k < n)
        def _():
            in_copy(i + k, slot).start()

        return carry

    jax.lax.fori_loop(0, n, body, 0)

    for i in range(max(0, n - k), n):
        out_copy(i, i % k).wait()


def kernel(x, w1, w2):
    B, C, D, H, W = x.shape
    Cr = w1.shape[1]
    S = D * H * W
    xf = x.reshape(B, C, S)

    TB = 1
    K = 6
    n = B // TB

    out = pl.pallas_call(
        functools.partial(_se_ring_kernel, n=n, k=K, tb=TB, inv_s=1.0 / float(S)),
        out_shape=jax.ShapeDtypeStruct((B, C, S), x.dtype),
        in_specs=[
            pl.BlockSpec(memory_space=pltpu.HBM),
            pl.BlockSpec(memory_space=pltpu.VMEM),
            pl.BlockSpec(memory_space=pltpu.VMEM),
        ],
        out_specs=pl.BlockSpec(memory_space=pltpu.HBM),
        scratch_shapes=[
            pltpu.VMEM((K, TB, C, S), x.dtype),
            pltpu.VMEM((K, TB, C, S), x.dtype),
            pltpu.SemaphoreType.DMA((K,)),
            pltpu.SemaphoreType.DMA((K,)),
        ],
        compiler_params=pltpu.CompilerParams(
            vmem_limit_bytes=60 * 1024 * 1024),
    )(xf, w1, w2)

    return out.reshape(B, C, D, H, W)


# speedup vs baseline: 1.3186x; 1.3186x over previous
import functools

import jax
import jax.numpy as jnp
from jax.experimental import pallas as pl
from jax.experimental.pallas import tpu as pltpu


def _se_gate_kernel(x_ref, w1_ref, w2_ref, g_ref, *, inv_s):
    se = jnp.sum(x_ref[...].astype(jnp.float32), axis=-1) * inv_s
    h = jnp.maximum(
        jnp.dot(se, w1_ref[...].astype(jnp.float32),
                preferred_element_type=jnp.float32), 0.0)
    g = jax.nn.sigmoid(
        jnp.dot(h, w2_ref[...].astype(jnp.float32),
                preferred_element_type=jnp.float32))
    g_ref[...] = g[:, None, :]


def kernel(x, w1, w2):
    B, C, D, H, W = x.shape
    Cr = w1.shape[1]
    S = D * H * W
    xf = x.reshape(B, C, S)

    TB = 2 if B % 2 == 0 else 1
    grid = (B // TB,)

    g = pl.pallas_call(
        functools.partial(_se_gate_kernel, inv_s=1.0 / float(S)),
        out_shape=jax.ShapeDtypeStruct((B, 1, C), jnp.float32),
        grid=grid,
        in_specs=[
            pl.BlockSpec((TB, C, S), lambda b: (b, 0, 0)),
            pl.BlockSpec((C, Cr), lambda b: (0, 0)),
            pl.BlockSpec((Cr, C), lambda b: (0, 0)),
        ],
        out_specs=pl.BlockSpec((TB, 1, C), lambda b: (b, 0, 0)),
        compiler_params=pltpu.CompilerParams(
            dimension_semantics=("arbitrary",),
            vmem_limit_bytes=56 * 1024 * 1024),
    )(xf, w1, w2)

    gate = g.reshape(B, C).astype(x.dtype)
    return x * gate[:, :, None, None, None]
